# Initial kernel scaffold; baseline (speedup 1.0000x reference)
import jax
import jax.numpy as jnp
from jax.experimental import pallas as pl


def kernel(x, edge_index, U, V):
    raise NotImplementedError("write your pallas kernel here")


# speedup vs baseline: 4.3227x; 4.3227x over previous
import functools

import jax
import jax.numpy as jnp
from jax import lax
from jax.experimental import pallas as pl
from jax.experimental.pallas import tpu as pltpu
from jax.experimental.pallas import tpu_sc as plsc

NC = 2
NS = 16
NW = NC * NS
K = 128


def _acc_rows(n_nodes):
  return ((n_nodes + 1 + NS * 8 - 1) // (NS * 8)) * (NS * 8)


def _sc_segment_sum(n_nodes, d, ch):
  np_rows = _acc_rows(n_nodes)
  zrows = np_rows // NS

  mesh = plsc.VectorSubcoreMesh(
      core_axis_name="c", subcore_axis_name="s", num_cores=NC,
      num_subcores=NS)

  @functools.partial(
      pl.kernel,
      out_type=jax.ShapeDtypeStruct((NC, np_rows, d), jnp.float32),
      mesh=mesh,
      scratch_types=[
          pltpu.VMEM((ch, K), jnp.int32),
          pltpu.VMEM((ch, K), jnp.int32),
          pltpu.VMEM((K, d), jnp.float32),
          pltpu.VMEM_SHARED((np_rows, d), jnp.float32),
          pltpu.SemaphoreType.DMA,
      ],
  )
  def seg_sum(x_hbm, src_hbm, dst_hbm, z_hbm, out_hbm,
              src_v, dst_v, rows_v, agg_sh, sem):
    c = lax.axis_index("c")
    s = lax.axis_index("s")
    wid = c * NS + s

    pltpu.sync_copy(z_hbm, agg_sh.at[pl.ds(s * zrows, zrows)])
    pltpu.sync_copy(src_hbm.at[wid], src_v)
    pltpu.sync_copy(dst_hbm.at[wid], dst_v)
    plsc.subcore_barrier()

    def body(j, carry):
      pltpu.async_copy(x_hbm.at[src_v.at[j]], rows_v, sem).wait()
      pltpu.sync_copy(rows_v, agg_sh.at[dst_v.at[j]], add=True)
      return carry

    lax.fori_loop(0, ch, body, 0)
    plsc.subcore_barrier()

    r0 = s * zrows
    pltpu.sync_copy(agg_sh.at[pl.ds(r0, zrows)],
                    out_hbm.at[c].at[pl.ds(r0, zrows)])

  return seg_sum


def _tc_finish_body(x_ref, agg_ref, u_ref, v_ref, o_ref):
  agg = agg_ref[0] + agg_ref[1]
  h = (jnp.dot(x_ref[...], u_ref[...], preferred_element_type=jnp.float32)
       + jnp.dot(agg, v_ref[...], preferred_element_type=jnp.float32))
  h = jnp.maximum(h, 0.0)
  m = jnp.max(h, axis=-1, keepdims=True)
  e = jnp.exp(h - m)
  o_ref[...] = e / jnp.sum(e, axis=-1, keepdims=True)


def kernel(x, edge_index, U, V):
  n, d = x.shape
  e = edge_index.shape[1]

  per_tile = (e + NW * K - 1) // (NW * K) * K
  ch = per_tile // K
  e_pad = per_tile * NW
  pad = e_pad - e
  src = jnp.concatenate([edge_index[0], jnp.zeros((pad,), jnp.int32)])
  dst = jnp.concatenate([edge_index[1], jnp.full((pad,), n, jnp.int32)])
  src = src.reshape(NW, ch, K)
  dst = dst.reshape(NW, ch, K)

  np_rows = _acc_rows(n)
  zeros = jnp.zeros((np_rows // NS, d), jnp.float32)

  agg2 = _sc_segment_sum(n, d, ch)(x, src, dst, zeros)

  blk = 1000
  grid = n // blk
  out = pl.pallas_call(
      _tc_finish_body,
      grid=(grid,),
      in_specs=[
          pl.BlockSpec((blk, d), lambda i: (i, 0)),
          pl.BlockSpec((NC, blk, d), lambda i: (0, i, 0)),
          pl.BlockSpec((d, d), lambda i: (0, 0)),
          pl.BlockSpec((d, d), lambda i: (0, 0)),
      ],
      out_specs=pl.BlockSpec((blk, d), lambda i: (i, 0)),
      out_shape=jax.ShapeDtypeStruct((n, d), jnp.float32),
  )(x, agg2, U, V)
  return out
